# baseline (speedup 1.0000x reference)
import jax
import jax.numpy as jnp
from jax.experimental import pallas as pl
from jax.experimental.pallas import tpu as pltpu

_BF = 512


def _routing(l_ref, b):
    B = l_ref.shape[0]
    fv = jnp.int32(0)
    l_fv = l_ref[0, 0]
    for r in range(B - 1, -1, -1):
        lr = l_ref[r, 0]
        fv = jnp.where(lr > 3, jnp.int32(r), fv)
        l_fv = jnp.where(lr > 3, lr, l_fv)
    lb = l_ref[b, 0]
    vb = lb > 3
    e_b = jnp.clip(jnp.where(vb, lb, l_fv) - 4, 0, jnp.int32(7))
    x_b = jnp.where(vb, b, fv)
    j_mul = jnp.where(vb, 1, 0)
    return e_b, x_b, j_mul


def _ffn_body(l_ref, x_ref, w1_ref, w3_ref, w2_ref, o_ref):
    b = pl.program_id(0)
    j = pl.program_id(1)
    cnt = jnp.float32(0.0)
    for k in range(l_ref.shape[1]):
        cnt = cnt + (l_ref[b, k] > 3).astype(jnp.float32)
    routing = jnp.where(cnt > 0.0, 1.0 / jnp.maximum(cnt, 1.0), 1.0)
    scale = routing * jnp.minimum(cnt, 1.0)

    @pl.when(jnp.logical_and(j == 0, scale == 0.0))
    def _zero():
        o_ref[...] = jnp.zeros_like(o_ref)

    @pl.when(scale != 0.0)
    def _compute():
        x = x_ref[0]
        a = jnp.dot(x, w1_ref[0], preferred_element_type=jnp.float32)
        c = jnp.dot(x, w3_ref[0], preferred_element_type=jnp.float32)
        gelu_a = 0.5 * a * (1.0 + jax.lax.erf(a * 0.7071067811865476))
        mid = (gelu_a * scale) * c
        val = jnp.dot(mid, w2_ref[0], preferred_element_type=jnp.float32)

        @pl.when(j == 0)
        def _set():
            o_ref[0] = val

        @pl.when(j != 0)
        def _acc():
            o_ref[0] += val


def _x_map(b, j, l):
    _, x_b, _ = _routing(l, b)
    return (x_b, 0, 0)


def _w13_map(b, j, l):
    e_b, _, j_mul = _routing(l, b)
    return (e_b, 0, j * j_mul)


def _w2_map(b, j, l):
    e_b, _, j_mul = _routing(l, b)
    return (e_b, j * j_mul, 0)


def kernel(hidden_states, W1, W2, W3, langs):
    B, S, D = hidden_states.shape
    E, _, F = W1.shape
    nj = F // _BF

    grid_spec = pltpu.PrefetchScalarGridSpec(
        num_scalar_prefetch=1,
        grid=(B, nj),
        in_specs=[
            pl.BlockSpec((1, S, D), _x_map),
            pl.BlockSpec((1, D, _BF), _w13_map),
            pl.BlockSpec((1, D, _BF), _w13_map),
            pl.BlockSpec((1, _BF, D), _w2_map),
        ],
        out_specs=pl.BlockSpec((1, S, D), lambda b, j, l: (b, 0, 0)),
    )
    return pl.pallas_call(
        _ffn_body,
        grid_spec=grid_spec,
        out_shape=jax.ShapeDtypeStruct((B, S, D), jnp.float32),
    )(langs, hidden_states, W1, W3, W2)


# speedup vs baseline: 1.0785x; 1.0785x over previous
import jax
import jax.numpy as jnp
from jax.experimental import pallas as pl
from jax.experimental.pallas import tpu as pltpu

_BF = 512


def _routing(l_ref, b):
    B = l_ref.shape[0]
    fv = jnp.int32(0)
    l_fv = l_ref[0, 0]
    for r in range(B - 1, -1, -1):
        lr = l_ref[r, 0]
        fv = jnp.where(lr > 3, jnp.int32(r), fv)
        l_fv = jnp.where(lr > 3, lr, l_fv)
    lb = l_ref[b, 0]
    vb = lb > 3
    e_b = jnp.clip(jnp.where(vb, lb, l_fv) - 4, 0, jnp.int32(7))
    x_b = jnp.where(vb, b, fv)
    j_mul = jnp.where(vb, 1, 0)
    return e_b, x_b, j_mul


def _ffn_body(l_ref, x_ref, w1_ref, w3_ref, w2_ref, o_ref):
    b = pl.program_id(0)
    j = pl.program_id(1)
    cnt = jnp.float32(0.0)
    for k in range(l_ref.shape[1]):
        cnt = cnt + (l_ref[b, k] > 3).astype(jnp.float32)
    routing = jnp.where(cnt > 0.0, 1.0 / jnp.maximum(cnt, 1.0), 1.0)
    scale = routing * jnp.minimum(cnt, 1.0)

    @pl.when(j == 0)
    def _zero():
        o_ref[...] = jnp.zeros_like(o_ref)

    @pl.when(scale != 0.0)
    def _compute():
        x = x_ref[0]
        a = jnp.dot(x, w1_ref[0], preferred_element_type=jnp.float32)
        c = jnp.dot(x, w3_ref[0], preferred_element_type=jnp.float32)
        gelu_a = 0.5 * a * (1.0 + jax.lax.erf(a * 0.7071067811865476))
        mid = (gelu_a * scale) * c
        o_ref[0] += jnp.dot(mid, w2_ref[0],
                            preferred_element_type=jnp.float32)


def _x_map(b, j, l):
    _, x_b, _ = _routing(l, b)
    return (x_b, 0, 0)


def _w13_map(b, j, l):
    e_b, _, j_mul = _routing(l, b)
    return (e_b, 0, j * j_mul)


def _w2_map(b, j, l):
    e_b, _, j_mul = _routing(l, b)
    return (e_b, j * j_mul, 0)


def kernel(hidden_states, W1, W2, W3, langs):
    B, S, D = hidden_states.shape
    E, _, F = W1.shape
    nj = F // _BF

    grid_spec = pltpu.PrefetchScalarGridSpec(
        num_scalar_prefetch=1,
        grid=(B, nj),
        in_specs=[
            pl.BlockSpec((1, S, D), _x_map),
            pl.BlockSpec((1, D, _BF), _w13_map),
            pl.BlockSpec((1, D, _BF), _w13_map),
            pl.BlockSpec((1, _BF, D), _w2_map),
        ],
        out_specs=pl.BlockSpec((1, S, D), lambda b, j, l: (b, 0, 0)),
    )
    return pl.pallas_call(
        _ffn_body,
        grid_spec=grid_spec,
        out_shape=jax.ShapeDtypeStruct((B, S, D), jnp.float32),
    )(langs, hidden_states, W1, W3, W2)
